# baseline (speedup 1.0000x reference)
import jax
import jax.numpy as jnp
from jax import lax
from jax.experimental import pallas as pl
from jax.experimental.pallas import tpu as pltpu


def _embed_packed_kernel(x_ref, tokT_ref, time_ref, o_ref):
    bb, t = x_ref.shape
    d, v = tokT_ref.shape
    k = o_ref.shape[2] // d
    tp = t // k

    row = lax.broadcasted_iota(jnp.int32, (v, t), 0)
    tokT = tokT_ref[...]
    timeT = time_ref[...]
    for bi in range(bb):
        xb = jnp.broadcast_to(x_ref[bi:bi + 1, :], (v, t))
        hotT = (xb == row).astype(jnp.bfloat16)
        outT = jnp.dot(tokT, hotT,
                       preferred_element_type=jnp.float32)
        outT = outT + timeT
        for j in range(k):
            o_ref[bi, :, j * d:(j + 1) * d] = jnp.transpose(
                outT[:, j * tp:(j + 1) * tp])


def _embed_direct_kernel(x_ref, tokT_ref, time_ref, o_ref):
    bb, t = x_ref.shape
    d, v = tokT_ref.shape
    row = lax.broadcasted_iota(jnp.int32, (v, t), 0)
    tokT = tokT_ref[...]
    timeT = time_ref[...]
    for bi in range(bb):
        xb = jnp.broadcast_to(x_ref[bi:bi + 1, :], (v, t))
        hotT = (xb == row).astype(jnp.bfloat16)
        outT = jnp.dot(tokT, hotT, preferred_element_type=jnp.float32)
        o_ref[bi] = jnp.transpose(outT + timeT)


@jax.jit
def kernel(x, token_weight, time_weight):
    b, t = x.shape
    v, d = token_weight.shape
    tokT = jnp.transpose(token_weight).astype(jnp.bfloat16)
    timeT = time_weight[:, :t]

    bb = 8
    while b % bb:
        bb //= 2
    params = pltpu.CompilerParams(dimension_semantics=("parallel",))

    k = 128 // d if (d < 128 and 128 % d == 0) else 1
    if k > 1 and t % k == 0:
        tp = t // k
        xp = jnp.concatenate([x[:, j::k] for j in range(k)], axis=1)
        timeP = jnp.concatenate([timeT[:, j::k] for j in range(k)], axis=1)
        out = pl.pallas_call(
            _embed_packed_kernel,
            out_shape=jax.ShapeDtypeStruct((b, tp, k * d),
                                           token_weight.dtype),
            grid_spec=pltpu.PrefetchScalarGridSpec(
                num_scalar_prefetch=0,
                grid=(b // bb,),
                in_specs=[
                    pl.BlockSpec((bb, t), lambda i: (i, 0)),
                    pl.BlockSpec((d, v), lambda i: (0, 0),
                                 pipeline_mode=pl.Buffered(1)),
                    pl.BlockSpec((d, t), lambda i: (0, 0),
                                 pipeline_mode=pl.Buffered(1)),
                ],
                out_specs=pl.BlockSpec((bb, tp, k * d),
                                       lambda i: (i, 0, 0)),
            ),
            compiler_params=params,
        )(xp.astype(jnp.int32), tokT, timeP)
        return out.reshape(b, t, d)

    out = pl.pallas_call(
        _embed_direct_kernel,
        out_shape=jax.ShapeDtypeStruct((b, t, d), token_weight.dtype),
        grid_spec=pltpu.PrefetchScalarGridSpec(
            num_scalar_prefetch=0,
            grid=(b // bb,),
            in_specs=[
                pl.BlockSpec((bb, t), lambda i: (i, 0)),
                pl.BlockSpec((d, v), lambda i: (0, 0),
                             pipeline_mode=pl.Buffered(1)),
                pl.BlockSpec((d, t), lambda i: (0, 0),
                             pipeline_mode=pl.Buffered(1)),
            ],
            out_specs=pl.BlockSpec((bb, t, d), lambda i: (i, 0, 0)),
        ),
        compiler_params=params,
    )(x.astype(jnp.int32), tokT, timeT)
    return out


# speedup vs baseline: 8.7232x; 8.7232x over previous
import jax
import jax.numpy as jnp
from jax import lax
from jax.experimental import pallas as pl
from jax.experimental.pallas import tpu as pltpu


def _embed_t_kernel(x_ref, tokT_ref, time_ref, o_ref):
    bb, t = x_ref.shape
    d, v = tokT_ref.shape

    row = lax.broadcasted_iota(jnp.int32, (v, t), 0)
    tokT = tokT_ref[...]
    timeT = time_ref[...]
    for bi in range(bb):
        xb = jnp.broadcast_to(x_ref[bi:bi + 1, :], (v, t))
        hotT = (xb == row).astype(jnp.bfloat16)
        outT = jnp.dot(tokT, hotT,
                       preferred_element_type=jnp.float32)
        o_ref[bi] = outT + timeT


@jax.jit
def kernel(x, token_weight, time_weight):
    b, t = x.shape
    v, d = token_weight.shape
    tokT = jnp.transpose(token_weight).astype(jnp.bfloat16)
    timeT = time_weight[:, :t]

    bb = 8
    while b % bb:
        bb //= 2

    outT = pl.pallas_call(
        _embed_t_kernel,
        out_shape=jax.ShapeDtypeStruct((b, d, t), token_weight.dtype),
        grid_spec=pltpu.PrefetchScalarGridSpec(
            num_scalar_prefetch=0,
            grid=(b // bb,),
            in_specs=[
                pl.BlockSpec((bb, t), lambda i: (i, 0)),
                pl.BlockSpec((d, v), lambda i: (0, 0),
                             pipeline_mode=pl.Buffered(1)),
                pl.BlockSpec((d, t), lambda i: (0, 0),
                             pipeline_mode=pl.Buffered(1)),
            ],
            out_specs=pl.BlockSpec((bb, d, t), lambda i: (i, 0, 0)),
        ),
        compiler_params=pltpu.CompilerParams(
            dimension_semantics=("parallel",)),
    )(x.astype(jnp.int32), tokT, timeT)
    return jnp.transpose(outT, (0, 2, 1))


# speedup vs baseline: 9.8316x; 1.1271x over previous
import jax
import jax.numpy as jnp
from jax import lax
from jax.experimental import pallas as pl
from jax.experimental.pallas import tpu as pltpu


def _embed_t_kernel(x_ref, tokT_ref, time_ref, o_ref):
    bb, t = x_ref.shape
    d, v = tokT_ref.shape

    row = lax.broadcasted_iota(jnp.int32, (v, t), 0)
    tokT = tokT_ref[...]
    timeT = time_ref[...]
    for bi in range(bb):
        xb = jnp.broadcast_to(x_ref[bi:bi + 1, :], (v, t))
        hotT = (xb == row).astype(jnp.bfloat16)
        outT = jnp.dot(tokT, hotT,
                       preferred_element_type=jnp.float32)
        o_ref[bi] = outT + timeT


@jax.jit
def kernel(x, token_weight, time_weight):
    b, t = x.shape
    v, d = token_weight.shape
    tokT = jnp.transpose(token_weight).astype(jnp.bfloat16)
    timeT = time_weight[:, :t]

    bb = 16
    while b % bb:
        bb //= 2

    outT = pl.pallas_call(
        _embed_t_kernel,
        out_shape=jax.ShapeDtypeStruct((b, d, t), token_weight.dtype),
        grid_spec=pltpu.PrefetchScalarGridSpec(
            num_scalar_prefetch=0,
            grid=(b // bb,),
            in_specs=[
                pl.BlockSpec((bb, t), lambda i: (i, 0)),
                pl.BlockSpec((d, v), lambda i: (0, 0),
                             pipeline_mode=pl.Buffered(1)),
                pl.BlockSpec((d, t), lambda i: (0, 0),
                             pipeline_mode=pl.Buffered(1)),
            ],
            out_specs=pl.BlockSpec((bb, d, t), lambda i: (i, 0, 0)),
        ),
        compiler_params=pltpu.CompilerParams(
            dimension_semantics=("parallel",)),
    )(x.astype(jnp.int32), tokT, timeT)
    return jnp.transpose(outT, (0, 2, 1))


# speedup vs baseline: 9.9059x; 1.0076x over previous
import jax
import jax.numpy as jnp
from jax import lax
from jax.experimental import pallas as pl
from jax.experimental.pallas import tpu as pltpu


def _embed_t_kernel(x_ref, tokT_ref, time_ref, o_ref):
    bb, t = x_ref.shape
    d, v = tokT_ref.shape

    row = lax.broadcasted_iota(jnp.int32, (v, t), 0)
    tokT = tokT_ref[...]
    timeT = time_ref[...]
    for bi in range(bb):
        xb = jnp.broadcast_to(x_ref[bi:bi + 1, :], (v, t))
        hotT = (xb == row).astype(jnp.bfloat16)
        outT = jnp.dot(tokT, hotT,
                       preferred_element_type=jnp.float32)
        o_ref[bi] = outT + timeT


@jax.jit
def kernel(x, token_weight, time_weight):
    b, t = x.shape
    v, d = token_weight.shape
    tokT = jnp.transpose(token_weight).astype(jnp.bfloat16)
    timeT = time_weight[:, :t]

    bb = 32
    while b % bb:
        bb //= 2

    outT = pl.pallas_call(
        _embed_t_kernel,
        out_shape=jax.ShapeDtypeStruct((b, d, t), token_weight.dtype),
        grid_spec=pltpu.PrefetchScalarGridSpec(
            num_scalar_prefetch=0,
            grid=(b // bb,),
            in_specs=[
                pl.BlockSpec((bb, t), lambda i: (i, 0)),
                pl.BlockSpec((d, v), lambda i: (0, 0),
                             pipeline_mode=pl.Buffered(1)),
                pl.BlockSpec((d, t), lambda i: (0, 0),
                             pipeline_mode=pl.Buffered(1)),
            ],
            out_specs=pl.BlockSpec((bb, d, t), lambda i: (i, 0, 0)),
        ),
        compiler_params=pltpu.CompilerParams(
            dimension_semantics=("parallel",)),
    )(x.astype(jnp.int32), tokT, timeT)
    return jnp.transpose(outT, (0, 2, 1))
